# baseline (speedup 1.0000x reference)
import jax
import jax.numpy as jnp
from jax import lax
from jax.experimental import pallas as pl
from jax.experimental.pallas import tpu as pltpu
from jax.experimental.pallas import tpu_sc as plsc

N = 10000
NP = 10240
D = 128
DH = D // 2
NC = 2
NS = 16
B = 128
PASS = 40
NPASS = 4
NB = PASS * NPASS
ROWS_PER_TILE = NP // NS


def _bcast_lane(v16, lane):
    idx = jnp.full((16, 1), lane, dtype=jnp.int32)
    dn = lax.GatherDimensionNumbers(
        offset_dims=(), collapsed_slice_dims=(0,), start_index_map=(0,))
    return lax.gather(v16, idx, dn, slice_sizes=(1,),
                      mode=lax.GatherScatterMode.PROMISE_IN_BOUNDS)


DW = DH // 2


def _sc_spmv_partial(xh, col, row, val):
    mesh = plsc.VectorSubcoreMesh(core_axis_name="c", subcore_axis_name="s")

    def body(x_hbm, col_hbm, row_hbm, val_hbm, out_hbm,
             colbuf, rowbuf, valbuf,
             gbuf0, gbuf1, sbuf0, sbuf1, xsp, acc_sh,
             gsem0, gsem1, ssem0, ssem1):
        gbufs = (gbuf0, gbuf1)
        sbufs = (sbuf0, sbuf1)
        gsems = (gsem0, gsem1)
        ssems = (ssem0, ssem1)
        cid = lax.axis_index("c")
        sid = lax.axis_index("s")
        off = sid * ROWS_PER_TILE

        @pl.when(sid < NS - 1)
        def _():
            pltpu.sync_copy(
                x_hbm.at[pl.ds(off, ROWS_PER_TILE), pl.ds(cid * DW, DW)],
                xsp.at[pl.ds(off, ROWS_PER_TILE)])

        @pl.when(sid == NS - 1)
        def _():
            pltpu.sync_copy(
                x_hbm.at[pl.ds(off, N - (NS - 1) * ROWS_PER_TILE),
                         pl.ds(cid * DW, DW)],
                xsp.at[pl.ds(off, N - (NS - 1) * ROWS_PER_TILE)])

        def zbody(i, carry):
            for k in range(DH // 16):
                sbuf0[i, pl.ds(k * 16, 16)] = jnp.zeros((16,), jnp.float32)
            return carry
        lax.fori_loop(0, B, zbody, 0)
        for c in range(ROWS_PER_TILE // B):
            pltpu.sync_copy(sbuf0, acc_sh.at[pl.ds(off + c * B, B)])
        plsc.subcore_barrier()

        for ps in range(NPASS):
            sl = pl.ds(ps * PASS, PASS)
            pltpu.sync_copy(col_hbm.at[sid, sl], colbuf)
            pltpu.sync_copy(row_hbm.at[sid, sl], rowbuf)
            pltpu.sync_copy(val_hbm.at[sid, sl], valbuf)

            for p in range(2):
                pltpu.async_copy(xsp.at[colbuf.at[p]], gbufs[p], gsems[p])

            def outer(t, carry):
                for p in range(2):
                    i = 2 * t + p
                    gb, sb = gbufs[p], sbufs[p]
                    pltpu.make_async_copy(xsp.at[colbuf.at[i]], gb,
                                          gsems[p]).wait()

                    @pl.when(jnp.logical_or(t > 0, ps > 0))
                    def _():
                        pltpu.make_async_copy(
                            sb, acc_sh.at[rowbuf.at[i]], ssems[p]).wait()

                    def jbody(j, c2):
                        v16 = valbuf[i, pl.ds(j * 16, 16)]
                        msk = jnp.full((16,), -65536, jnp.int32)
                        for e in range(16):
                            bv = _bcast_lane(v16, e)
                            r = j * 16 + e
                            for k in range(DH // 32):
                                vw = gb[r, pl.ds(k * 16, 16)]
                                va = plsc.bitcast(
                                    lax.shift_left(vw, 16), jnp.float32)
                                vb = plsc.bitcast(
                                    lax.bitwise_and(vw, msk), jnp.float32)
                                sb[r, pl.ds(k * 32, 16)] = va * bv
                                sb[r, pl.ds(k * 32 + 16, 16)] = vb * bv
                        return c2
                    lax.fori_loop(0, B // 16, jbody, 0)

                    pltpu.async_copy(sb, acc_sh.at[rowbuf.at[i]], ssems[p],
                                     add=True)

                    @pl.when(t < PASS // 2 - 1)
                    def _():
                        pltpu.async_copy(xsp.at[colbuf.at[i + 2]], gb,
                                         gsems[p])
                return carry
            lax.fori_loop(0, PASS // 2, outer, 0)

        for p in range(2):
            pltpu.make_async_copy(sbufs[p], acc_sh.at[rowbuf.at[PASS - 2 + p]],
                                  ssems[p]).wait()

        plsc.subcore_barrier()

        @pl.when(sid < NS - 1)
        def _():
            pltpu.sync_copy(
                acc_sh.at[pl.ds(off, ROWS_PER_TILE)],
                out_hbm.at[pl.ds(off, ROWS_PER_TILE), pl.ds(cid * DH, DH)])

        @pl.when(sid == NS - 1)
        def _():
            pltpu.sync_copy(
                acc_sh.at[pl.ds(off, N - (NS - 1) * ROWS_PER_TILE)],
                out_hbm.at[pl.ds(off, N - (NS - 1) * ROWS_PER_TILE),
                           pl.ds(cid * DH, DH)])

    run = pl.kernel(
        body,
        mesh=mesh,
        compiler_params=pltpu.CompilerParams(use_tc_tiling_on_sc=False,
                                             needs_layout_passes=False),
        out_type=jax.ShapeDtypeStruct((N, D), jnp.float32),
        scratch_types=[
            pltpu.VMEM((PASS, B), jnp.int32),
            pltpu.VMEM((PASS, B), jnp.int32),
            pltpu.VMEM((PASS, B), jnp.float32),
            pltpu.VMEM((B, DW), jnp.int32),
            pltpu.VMEM((B, DW), jnp.int32),
            pltpu.VMEM((B, DH), jnp.float32),
            pltpu.VMEM((B, DH), jnp.float32),
            pltpu.VMEM_SHARED((NP, DW), jnp.int32),
            pltpu.VMEM_SHARED((NP, DH), jnp.float32),
            pltpu.SemaphoreType.DMA,
            pltpu.SemaphoreType.DMA,
            pltpu.SemaphoreType.DMA,
            pltpu.SemaphoreType.DMA,
        ],
    )
    return run(xh, col, row, val)


def _tc_pad_edges(row, col, val, e_pad):
    e = val.shape[0]

    def body(r_ref, c_ref, v_ref, ro_ref, co_ref, vo_ref):
        for src, dst in ((r_ref, ro_ref), (c_ref, co_ref), (v_ref, vo_ref)):
            dst[pl.ds(0, e)] = src[...]
            dst[pl.ds(e, e_pad - e)] = jnp.zeros((e_pad - e,), src.dtype)

    return pl.pallas_call(
        body,
        out_shape=(jax.ShapeDtypeStruct((e_pad,), jnp.int32),
                   jax.ShapeDtypeStruct((e_pad,), jnp.int32),
                   jax.ShapeDtypeStruct((e_pad,), jnp.float32)),
    )(row, col, val)


@jax.jit
def kernel(x, A_ind, A_val):
    row = A_ind[0].astype(jnp.int32)
    col = A_ind[1].astype(jnp.int32)
    val = A_val.astype(jnp.float32)
    e = val.shape[0]
    chunk = NS * B * 2 * NPASS
    e_pad = ((e + chunk - 1) // chunk) * chunk
    if e_pad != e:
        row, col, val = _tc_pad_edges(row, col, val, e_pad)
    nb = e_pad // (NS * B)
    assert nb == NB, (nb, NB)
    col3 = col.reshape(NS, NB, B)
    row3 = row.reshape(NS, NB, B)
    val3 = val.reshape(NS, NB, B)
    perm = []
    for p in range(D):
        g, q = p // 32, p % 32
        perm.append(32 * g + (q // 2 if q % 2 == 0 else 16 + q // 2))
    xbf = x[:, jnp.array(perm, dtype=jnp.int32)].astype(jnp.bfloat16)
    xi = lax.bitcast_convert_type(
        xbf.reshape(N, D // 2, 2), jnp.int32)
    return _sc_spmv_partial(xi, col3, row3, val3)


# speedup vs baseline: 1.2658x; 1.2658x over previous
import jax
import jax.numpy as jnp
from jax import lax
from jax.experimental import pallas as pl
from jax.experimental.pallas import tpu as pltpu
from jax.experimental.pallas import tpu_sc as plsc

N = 10000
NP = 10240
D = 128
DH = D // 2
NC = 2
NS = 16
B = 128
PASS = 40
NPASS = 4
NB = PASS * NPASS
ROWS_PER_TILE = NP // NS


def _bcast_lane(v16, lane):
    idx = jnp.full((16, 1), lane, dtype=jnp.int32)
    dn = lax.GatherDimensionNumbers(
        offset_dims=(), collapsed_slice_dims=(0,), start_index_map=(0,))
    return lax.gather(v16, idx, dn, slice_sizes=(1,),
                      mode=lax.GatherScatterMode.PROMISE_IN_BOUNDS)


def _sc_spmv_partial(xh, col, row, val):
    mesh = plsc.VectorSubcoreMesh(core_axis_name="c", subcore_axis_name="s")

    def body(x_hbm, xh2_hbm, col_hbm, row_hbm, val_hbm, out_hbm,
             colbuf, rowbuf, valbuf,
             gbuf0, gbuf1, sbuf0, sbuf1, xsp, acc_sh,
             gsem0, gsem1, ssem0, ssem1):
        gbufs = (gbuf0, gbuf1)
        sbufs = (sbuf0, sbuf1)
        gsems = (gsem0, gsem1)
        ssems = (ssem0, ssem1)
        cid = lax.axis_index("c")
        sid = lax.axis_index("s")
        off = sid * ROWS_PER_TILE
        xh = xh2_hbm.at[cid]

        @pl.when(sid < NS - 1)
        def _():
            pltpu.sync_copy(
                x_hbm.at[pl.ds(off, ROWS_PER_TILE), pl.ds(cid * DH, DH)],
                xsp.at[pl.ds(off, ROWS_PER_TILE)])

        @pl.when(sid == NS - 1)
        def _():
            pltpu.sync_copy(
                x_hbm.at[pl.ds(off, N - (NS - 1) * ROWS_PER_TILE),
                         pl.ds(cid * DH, DH)],
                xsp.at[pl.ds(off, N - (NS - 1) * ROWS_PER_TILE)])

        def zbody(i, carry):
            for k in range(DH // 16):
                sbuf0[i, pl.ds(k * 16, 16)] = jnp.zeros((16,), jnp.float32)
            return carry
        lax.fori_loop(0, B, zbody, 0)
        for c in range(ROWS_PER_TILE // B):
            pltpu.sync_copy(sbuf0, acc_sh.at[pl.ds(off + c * B, B)])
        plsc.subcore_barrier()

        for ps in range(NPASS):
            sl = pl.ds(ps * PASS, PASS)
            pltpu.sync_copy(col_hbm.at[sid, sl], colbuf)
            pltpu.sync_copy(row_hbm.at[sid, sl], rowbuf)
            pltpu.sync_copy(val_hbm.at[sid, sl], valbuf)

            for p in range(2):
                pltpu.async_copy(xsp.at[colbuf.at[p]], gbufs[p], gsems[p])

            def outer(t, carry):
                for p in range(2):
                    i = 2 * t + p
                    gb, sb = gbufs[p], sbufs[p]
                    pltpu.make_async_copy(xsp.at[colbuf.at[i]], gb,
                                          gsems[p]).wait()

                    @pl.when(jnp.logical_or(t > 0, ps > 0))
                    def _():
                        pltpu.make_async_copy(
                            sb, acc_sh.at[rowbuf.at[i]], ssems[p]).wait()

                    def jbody(j, c2):
                        v16 = valbuf[i, pl.ds(j * 16, 16)]
                        for e in range(16):
                            bv = _bcast_lane(v16, e)
                            r = j * 16 + e
                            for k in range(DH // 16):
                                sb[r, pl.ds(k * 16, 16)] = (
                                    gb[r, pl.ds(k * 16, 16)] * bv)
                        return c2
                    lax.fori_loop(0, B // 16, jbody, 0)

                    pltpu.async_copy(sb, acc_sh.at[rowbuf.at[i]], ssems[p],
                                     add=True)

                    hbm_src = (t + 1) % 3 == 2

                    @pl.when(jnp.logical_and(t < PASS // 2 - 1,
                                             jnp.logical_not(hbm_src)))
                    def _():
                        pltpu.async_copy(xsp.at[colbuf.at[i + 2]], gb,
                                         gsems[p])

                    @pl.when(jnp.logical_and(t < PASS // 2 - 1, hbm_src))
                    def _():
                        pltpu.async_copy(xh.at[colbuf.at[i + 2]], gb,
                                         gsems[p])
                return carry
            lax.fori_loop(0, PASS // 2, outer, 0)

        for p in range(2):
            pltpu.make_async_copy(sbufs[p], acc_sh.at[rowbuf.at[PASS - 2 + p]],
                                  ssems[p]).wait()

        plsc.subcore_barrier()

        @pl.when(sid < NS - 1)
        def _():
            pltpu.sync_copy(
                acc_sh.at[pl.ds(off, ROWS_PER_TILE)],
                out_hbm.at[pl.ds(off, ROWS_PER_TILE), pl.ds(cid * DH, DH)])

        @pl.when(sid == NS - 1)
        def _():
            pltpu.sync_copy(
                acc_sh.at[pl.ds(off, N - (NS - 1) * ROWS_PER_TILE)],
                out_hbm.at[pl.ds(off, N - (NS - 1) * ROWS_PER_TILE),
                           pl.ds(cid * DH, DH)])

    run = pl.kernel(
        body,
        mesh=mesh,
        compiler_params=pltpu.CompilerParams(use_tc_tiling_on_sc=False),
        out_type=jax.ShapeDtypeStruct((N, D), jnp.float32),
        scratch_types=[
            pltpu.VMEM((PASS, B), jnp.int32),
            pltpu.VMEM((PASS, B), jnp.int32),
            pltpu.VMEM((PASS, B), jnp.float32),
            pltpu.VMEM((B, DH), jnp.float32),
            pltpu.VMEM((B, DH), jnp.float32),
            pltpu.VMEM((B, DH), jnp.float32),
            pltpu.VMEM((B, DH), jnp.float32),
            pltpu.VMEM_SHARED((NP, DH), jnp.float32),
            pltpu.VMEM_SHARED((NP, DH), jnp.float32),
            pltpu.SemaphoreType.DMA,
            pltpu.SemaphoreType.DMA,
            pltpu.SemaphoreType.DMA,
            pltpu.SemaphoreType.DMA,
        ],
    )
    return run(xh, jnp.stack([xh[:, :DH], xh[:, DH:]]), col, row, val)


def _tc_pad_edges(row, col, val, e_pad):
    e = val.shape[0]

    def body(r_ref, c_ref, v_ref, ro_ref, co_ref, vo_ref):
        for src, dst in ((r_ref, ro_ref), (c_ref, co_ref), (v_ref, vo_ref)):
            dst[pl.ds(0, e)] = src[...]
            dst[pl.ds(e, e_pad - e)] = jnp.zeros((e_pad - e,), src.dtype)

    return pl.pallas_call(
        body,
        out_shape=(jax.ShapeDtypeStruct((e_pad,), jnp.int32),
                   jax.ShapeDtypeStruct((e_pad,), jnp.int32),
                   jax.ShapeDtypeStruct((e_pad,), jnp.float32)),
    )(row, col, val)


@jax.jit
def kernel(x, A_ind, A_val):
    row = A_ind[0].astype(jnp.int32)
    col = A_ind[1].astype(jnp.int32)
    val = A_val.astype(jnp.float32)
    e = val.shape[0]
    chunk = NS * B * 2 * NPASS
    e_pad = ((e + chunk - 1) // chunk) * chunk
    if e_pad != e:
        row, col, val = _tc_pad_edges(row, col, val, e_pad)
    nb = e_pad // (NS * B)
    assert nb == NB, (nb, NB)
    col3 = col.reshape(NS, NB, B)
    row3 = row.reshape(NS, NB, B)
    val3 = val.reshape(NS, NB, B)
    return _sc_spmv_partial(x, col3, row3, val3)


# speedup vs baseline: 1.7355x; 1.3710x over previous
import jax
import jax.numpy as jnp
from jax import lax
from jax.experimental import pallas as pl
from jax.experimental.pallas import tpu as pltpu
from jax.experimental.pallas import tpu_sc as plsc

N = 10000
NP = 10240
D = 128
DH = D // 2
NC = 2
NS = 16
B = 128
PASS = 40
NPASS = 4
NB = PASS * NPASS
ROWS_PER_TILE = NP // NS


def _bcast_lane(v16, lane):
    idx = jnp.full((16, 1), lane, dtype=jnp.int32)
    dn = lax.GatherDimensionNumbers(
        offset_dims=(), collapsed_slice_dims=(0,), start_index_map=(0,))
    return lax.gather(v16, idx, dn, slice_sizes=(1,),
                      mode=lax.GatherScatterMode.PROMISE_IN_BOUNDS)


def _sc_spmv_partial(xh, col, row, val):
    mesh = plsc.VectorSubcoreMesh(core_axis_name="c", subcore_axis_name="s")

    def body(x_hbm, col_hbm, row_hbm, val_hbm, out_hbm,
             colbuf, rowbuf, valbuf,
             gbuf0, gbuf1, sbuf0, sbuf1, xsp, acc_sh,
             gsem0, gsem1, ssem0, ssem1):
        gbufs = (gbuf0, gbuf1)
        sbufs = (sbuf0, sbuf1)
        gsems = (gsem0, gsem1)
        ssems = (ssem0, ssem1)
        cid = lax.axis_index("c")
        sid = lax.axis_index("s")
        off = sid * ROWS_PER_TILE

        @pl.when(sid < NS - 1)
        def _():
            pltpu.sync_copy(
                x_hbm.at[pl.ds(off, ROWS_PER_TILE), pl.ds(cid * DH, DH)],
                xsp.at[pl.ds(off, ROWS_PER_TILE)])

        @pl.when(sid == NS - 1)
        def _():
            pltpu.sync_copy(
                x_hbm.at[pl.ds(off, N - (NS - 1) * ROWS_PER_TILE),
                         pl.ds(cid * DH, DH)],
                xsp.at[pl.ds(off, N - (NS - 1) * ROWS_PER_TILE)])

        def zbody(i, carry):
            for k in range(DH // 16):
                sbuf0[i, pl.ds(k * 16, 16)] = jnp.zeros((16,), jnp.float32)
            return carry
        lax.fori_loop(0, B, zbody, 0)
        for c in range(ROWS_PER_TILE // B):
            pltpu.sync_copy(sbuf0, acc_sh.at[pl.ds(off + c * B, B)])
        plsc.subcore_barrier()

        for ps in range(NPASS):
            sl = pl.ds(ps * PASS, PASS)
            pltpu.sync_copy(col_hbm.at[sid, sl], colbuf)
            pltpu.sync_copy(row_hbm.at[sid, sl], rowbuf)
            pltpu.sync_copy(val_hbm.at[sid, sl], valbuf)

            for p in range(2):
                pltpu.async_copy(xsp.at[colbuf.at[p]], gbufs[p], gsems[p])

            def outer(t, carry):
                for p in range(2):
                    i = 2 * t + p
                    gb, sb = gbufs[p], sbufs[p]
                    pltpu.make_async_copy(xsp.at[colbuf.at[i]], gb,
                                          gsems[p]).wait()

                    @pl.when(jnp.logical_or(t > 0, ps > 0))
                    def _():
                        pltpu.make_async_copy(
                            sb, acc_sh.at[rowbuf.at[i]], ssems[p]).wait()

                    def jbody(j, c2):
                        v16 = valbuf[i, pl.ds(j * 16, 16)]
                        for e in range(16):
                            bv = _bcast_lane(v16, e)
                            r = j * 16 + e
                            for k in range(DH // 16):
                                sb[r, pl.ds(k * 16, 16)] = (
                                    gb[r, pl.ds(k * 16, 16)] * bv)
                        return c2
                    lax.fori_loop(0, B // 16, jbody, 0)

                    pltpu.async_copy(sb, acc_sh.at[rowbuf.at[i]], ssems[p],
                                     add=True)

                    @pl.when(t < PASS // 2 - 1)
                    def _():
                        pltpu.async_copy(xsp.at[colbuf.at[i + 2]], gb,
                                         gsems[p])
                return carry
            lax.fori_loop(0, PASS // 2, outer, 0)

        for p in range(2):
            pltpu.make_async_copy(sbufs[p], acc_sh.at[rowbuf.at[PASS - 2 + p]],
                                  ssems[p]).wait()

        plsc.subcore_barrier()

        @pl.when(sid < NS - 1)
        def _():
            pltpu.sync_copy(
                acc_sh.at[pl.ds(off, ROWS_PER_TILE)],
                out_hbm.at[pl.ds(off, ROWS_PER_TILE), pl.ds(cid * DH, DH)])

        @pl.when(sid == NS - 1)
        def _():
            pltpu.sync_copy(
                acc_sh.at[pl.ds(off, N - (NS - 1) * ROWS_PER_TILE)],
                out_hbm.at[pl.ds(off, N - (NS - 1) * ROWS_PER_TILE),
                           pl.ds(cid * DH, DH)])

    run = pl.kernel(
        body,
        mesh=mesh,
        compiler_params=pltpu.CompilerParams(use_tc_tiling_on_sc=False),
        out_type=jax.ShapeDtypeStruct((N, D), jnp.float32),
        scratch_types=[
            pltpu.VMEM((PASS, B), jnp.int32),
            pltpu.VMEM((PASS, B), jnp.int32),
            pltpu.VMEM((PASS, B), jnp.float32),
            pltpu.VMEM((B, DH), jnp.float32),
            pltpu.VMEM((B, DH), jnp.float32),
            pltpu.VMEM((B, DH), jnp.float32),
            pltpu.VMEM((B, DH), jnp.float32),
            pltpu.VMEM_SHARED((NP, DH), jnp.float32),
            pltpu.VMEM_SHARED((NP, DH), jnp.float32),
            pltpu.SemaphoreType.DMA,
            pltpu.SemaphoreType.DMA,
            pltpu.SemaphoreType.DMA,
            pltpu.SemaphoreType.DMA,
        ],
    )
    return run(xh, col, row, val)


def _tc_pad_edges(row, col, val, e_pad):
    e = val.shape[0]

    def body(r_ref, c_ref, v_ref, ro_ref, co_ref, vo_ref):
        for src, dst in ((r_ref, ro_ref), (c_ref, co_ref), (v_ref, vo_ref)):
            dst[pl.ds(0, e)] = src[...]
            dst[pl.ds(e, e_pad - e)] = jnp.zeros((e_pad - e,), src.dtype)

    return pl.pallas_call(
        body,
        out_shape=(jax.ShapeDtypeStruct((e_pad,), jnp.int32),
                   jax.ShapeDtypeStruct((e_pad,), jnp.int32),
                   jax.ShapeDtypeStruct((e_pad,), jnp.float32)),
    )(row, col, val)


@jax.jit
def kernel(x, A_ind, A_val):
    row = A_ind[0].astype(jnp.int32)
    col = A_ind[1].astype(jnp.int32)
    val = A_val.astype(jnp.float32)
    e = val.shape[0]
    chunk = NS * B * 2 * NPASS
    e_pad = ((e + chunk - 1) // chunk) * chunk
    if e_pad != e:
        row, col, val = _tc_pad_edges(row, col, val, e_pad)
    nb = e_pad // (NS * B)
    assert nb == NB, (nb, NB)
    col3 = col.reshape(NS, NB, B)
    row3 = row.reshape(NS, NB, B)
    val3 = val.reshape(NS, NB, B)
    return _sc_spmv_partial(x, col3, row3, val3)
